# baseline (speedup 1.0000x reference)
import jax
import jax.numpy as jnp
from jax.experimental import pallas as pl

_B = 256
_DIN = 1024
_DF = 2048
_C = 1000
_K = 100
_N = _C + _B
_NP = 1280

_HI = jax.lax.Precision.HIGHEST


def _feat_kernel(x_ref, wt_ref, b_ref, z_ref):
    z_ref[...] = (
        jnp.dot(x_ref[...].astype(jnp.bfloat16), wt_ref[...].astype(jnp.bfloat16),
                preferred_element_type=jnp.float32)
        + b_ref[...]
    )


def _stats_kernel(s_ref, wt_ref, b_ref, ent_ref, yhat_ref, rsq_ref):
    s = s_ref[...]
    logits = (
        jnp.dot(s.astype(jnp.bfloat16), wt_ref[...].astype(jnp.bfloat16),
                preferred_element_type=jnp.float32)
        + b_ref[...]
    )
    m = jnp.max(logits, axis=1, keepdims=True)
    ex = jnp.exp(logits - m)
    se = jnp.sum(ex, axis=1, keepdims=True)
    logp = logits - m - jnp.log(se)
    ent_ref[...] = -jnp.sum(jnp.exp(logp) * logp, axis=1, keepdims=True)
    idx = jax.lax.broadcasted_iota(jnp.int32, logits.shape, 1)
    yhat_ref[...] = jnp.min(jnp.where(logits == m, idx, _C), axis=1, keepdims=True)
    rsq_ref[...] = jnp.sum(s * s, axis=1, keepdims=True)


def _sel_kernel(entc_ref, yc_ref, rsqc_ref, entr_ref, yr_ref, l_ref):
    pid = pl.program_id(0)
    blk = entc_ref.shape[0]
    entc = entc_ref[...]
    yc = yc_ref[...]
    entr = entr_ref[...]
    yr = yr_ref[...]
    i = pid * blk + jax.lax.broadcasted_iota(jnp.int32, (blk, _NP), 0)
    j = jax.lax.broadcasted_iota(jnp.int32, (blk, _NP), 1)
    same = (yc == yr) & (j < _N)
    earlier = (entr < entc) | ((entr == entc) & (j < i))
    rank = jnp.sum((same & earlier).astype(jnp.int32), axis=1, keepdims=True)
    icol = pid * blk + jax.lax.broadcasted_iota(jnp.int32, (blk, 1), 0)
    keep = (rank < _K) & (icol < _N)
    scale = jnp.where(
        keep, 1.0 / jnp.sqrt(jnp.maximum(rsqc_ref[...], 1e-24)), 0.0)
    cidx = jax.lax.broadcasted_iota(jnp.int32, (blk, _C), 1)
    l_ref[...] = jnp.where(cidx == yc, scale, 0.0)


def _weights_kernel(st_ref, l_ref, w_ref, cn_ref):
    wblk = jnp.dot(st_ref[...].astype(jnp.bfloat16), l_ref[...].astype(jnp.bfloat16),
                   preferred_element_type=jnp.float32)
    w_ref[...] = wblk

    @pl.when(pl.program_id(0) == 0)
    def _():
        cn_ref[...] = jnp.zeros_like(cn_ref)

    cn_ref[...] += jnp.sum(wblk * wblk, axis=0, keepdims=True)


def _out_kernel(z_ref, w_ref, cn_ref, o_ref):
    inv = 1.0 / jnp.sqrt(jnp.maximum(cn_ref[...], 1e-24))
    o_ref[...] = jnp.dot(z_ref[...].astype(jnp.bfloat16),
                         (w_ref[...] * inv).astype(jnp.bfloat16),
                         preferred_element_type=jnp.float32)


def kernel(x, feat_W, feat_b, cls_W, cls_b):
    fWT = feat_W.T
    cWT = cls_W.T
    fb = feat_b.reshape(1, _DF)
    cb = cls_b.reshape(1, _C)

    z = pl.pallas_call(
        _feat_kernel,
        out_shape=jax.ShapeDtypeStruct((_B, _DF), jnp.float32),
    )(x, fWT, fb)

    s_pad = jnp.concatenate(
        [cls_W, z, jnp.zeros((_NP - _N, _DF), jnp.float32)], axis=0)

    blk = 128
    grid = _NP // blk
    ent, yhat, rsq = pl.pallas_call(
        _stats_kernel,
        grid=(grid,),
        in_specs=[
            pl.BlockSpec((blk, _DF), lambda i: (i, 0)),
            pl.BlockSpec((_DF, _C), lambda i: (0, 0)),
            pl.BlockSpec((1, _C), lambda i: (0, 0)),
        ],
        out_specs=[
            pl.BlockSpec((blk, 1), lambda i: (i, 0)),
            pl.BlockSpec((blk, 1), lambda i: (i, 0)),
            pl.BlockSpec((blk, 1), lambda i: (i, 0)),
        ],
        out_shape=[
            jax.ShapeDtypeStruct((_NP, 1), jnp.float32),
            jax.ShapeDtypeStruct((_NP, 1), jnp.int32),
            jax.ShapeDtypeStruct((_NP, 1), jnp.float32),
        ],
    )(s_pad, cWT, cb)

    entr = ent.reshape(1, _NP)
    yr = yhat.reshape(1, _NP)

    lsel = pl.pallas_call(
        _sel_kernel,
        grid=(grid,),
        in_specs=[
            pl.BlockSpec((blk, 1), lambda i: (i, 0)),
            pl.BlockSpec((blk, 1), lambda i: (i, 0)),
            pl.BlockSpec((blk, 1), lambda i: (i, 0)),
            pl.BlockSpec((1, _NP), lambda i: (0, 0)),
            pl.BlockSpec((1, _NP), lambda i: (0, 0)),
        ],
        out_specs=pl.BlockSpec((blk, _C), lambda i: (i, 0)),
        out_shape=jax.ShapeDtypeStruct((_NP, _C), jnp.float32),
    )(ent, yhat, rsq, entr, yr)

    st_pad = jnp.concatenate(
        [cWT, z.T, jnp.zeros((_DF, _NP - _N), jnp.float32)], axis=1)

    dblk = 256
    weights, cn2 = pl.pallas_call(
        _weights_kernel,
        grid=(_DF // dblk,),
        in_specs=[
            pl.BlockSpec((dblk, _NP), lambda i: (i, 0)),
            pl.BlockSpec((_NP, _C), lambda i: (0, 0)),
        ],
        out_specs=[
            pl.BlockSpec((dblk, _C), lambda i: (i, 0)),
            pl.BlockSpec((1, _C), lambda i: (0, 0)),
        ],
        out_shape=[
            jax.ShapeDtypeStruct((_DF, _C), jnp.float32),
            jax.ShapeDtypeStruct((1, _C), jnp.float32),
        ],
    )(st_pad, lsel)

    out = pl.pallas_call(
        _out_kernel,
        out_shape=jax.ShapeDtypeStruct((_B, _C), jnp.float32),
    )(z, weights, cn2)
    return out
